# baseline (speedup 1.0000x reference)
import functools

import jax
import jax.numpy as jnp
from jax import lax
from jax.experimental import pallas as pl
from jax.experimental.pallas import tpu as pltpu
from jax.experimental.pallas import tpu_sc as plsc

_ROWS = 1000000
_DIM = 64
_NWORKERS = 32
_CHUNK = 248
_NCHUNKS = 126
_SPAN = _CHUNK * _NCHUNKS
_TAIL = _ROWS - _SPAN * _NWORKERS
_NBUF = 4


def _sc_copy(w_hbm, out_hbm, buf, in_sems, out_sems):
    wid = lax.axis_index("s") * 2 + lax.axis_index("c")
    base = wid * _SPAN

    def in_copy(c, b):
        return pltpu.make_async_copy(
            w_hbm.at[pl.ds(base + c * _CHUNK, _CHUNK), :],
            buf.at[b],
            in_sems.at[b],
        )

    def out_copy(c, b):
        return pltpu.make_async_copy(
            buf.at[b],
            out_hbm.at[pl.ds(base + c * _CHUNK, _CHUNK), :],
            out_sems.at[b],
        )

    in_copy(0, 0).start()
    in_copy(1, 1).start()
    for c in range(_NCHUNKS):
        b = c % _NBUF
        in_copy(c, b).wait()
        out_copy(c, b).start()
        j = c - 2
        if j >= 0:
            out_copy(j, j % _NBUF).wait()
        nxt = c + 2
        if 2 <= nxt < _NCHUNKS:
            in_copy(nxt, nxt % _NBUF).start()
    for j in range(max(0, _NCHUNKS - 2), _NCHUNKS):
        out_copy(j, j % _NBUF).wait()

    @pl.when(wid == _NWORKERS - 1)
    def _():
        tail_base = _SPAN * _NWORKERS
        tin = pltpu.make_async_copy(
            w_hbm.at[pl.ds(tail_base, _TAIL), :],
            buf.at[0, pl.ds(0, _TAIL), :],
            in_sems.at[0],
        )
        tin.start()
        tin.wait()
        tout = pltpu.make_async_copy(
            buf.at[0, pl.ds(0, _TAIL), :],
            out_hbm.at[pl.ds(tail_base, _TAIL), :],
            out_sems.at[0],
        )
        tout.start()
        tout.wait()


def kernel(weight):
    mesh = plsc.VectorSubcoreMesh(core_axis_name="c", subcore_axis_name="s")
    run = functools.partial(
        pl.kernel,
        mesh=mesh,
        out_type=jax.ShapeDtypeStruct((_ROWS, _DIM), jnp.float32),
        scratch_types=[
            pltpu.VMEM((_NBUF, _CHUNK, _DIM), jnp.float32),
            pltpu.SemaphoreType.DMA((_NBUF,)),
            pltpu.SemaphoreType.DMA((_NBUF,)),
        ],
        compiler_params=pltpu.CompilerParams(use_tc_tiling_on_sc=True),
    )(_sc_copy)
    return run(weight)


# speedup vs baseline: 6.6786x; 6.6786x over previous
import jax
import jax.numpy as jnp
from jax.experimental import pallas as pl
from jax.experimental.pallas import tpu as pltpu

_ROWS = 1000000
_DIM = 64
_BC = 32768


def _copy_block(in_ref, out_ref):
    out_ref[...] = in_ref[...]


def kernel(weight):
    wt = weight.T
    out_t = pl.pallas_call(
        _copy_block,
        grid=(pl.cdiv(_ROWS, _BC),),
        in_specs=[pl.BlockSpec((_DIM, _BC), lambda i: (0, i))],
        out_specs=pl.BlockSpec((_DIM, _BC), lambda i: (0, i)),
        out_shape=jax.ShapeDtypeStruct((_DIM, _ROWS), jnp.float32),
        compiler_params=pltpu.CompilerParams(
            dimension_semantics=("arbitrary",),
        ),
    )(wt)
    return out_t.T
